# baseline (speedup 1.0000x reference)
import functools

import jax
import jax.numpy as jnp
from jax import lax
from jax.experimental import pallas as pl
from jax.experimental.pallas import tpu as pltpu
from jax.experimental.pallas import tpu_sc as plsc

VOCAB = 100000
DIM = 128
BATCH = 4096
SEQ = 50

_INFO = plsc.get_sparse_core_info()
NC = _INFO.num_cores
NS = _INFO.num_subcores
NW = NC * NS

BPW = BATCH // NW
NSLOT = 2 * SEQ
NBUF = 4


@functools.partial(
    pl.kernel,
    out_type=jax.ShapeDtypeStruct((SEQ, BATCH, 2 * DIM), jnp.float32),
    mesh=plsc.VectorSubcoreMesh(core_axis_name="c", subcore_axis_name="s"),
    scratch_types=[
        pltpu.VMEM((SEQ, BPW), jnp.int32),
        pltpu.VMEM((SEQ, BPW), jnp.int32),
        pltpu.VMEM((NBUF, BPW, DIM), jnp.float32),
        pltpu.SemaphoreType.DMA,
        pltpu.SemaphoreType.DMA,
    ],
)
def _gather_rows(firsts_hbm, lasts_hbm, table_hbm, out_hbm,
                 firsts_v, lasts_v, rows_v, gsem, ssem):
    wid = lax.axis_index("s") * NC + lax.axis_index("c")
    b0 = wid * BPW
    pltpu.sync_copy(firsts_hbm.at[:, pl.ds(b0, BPW)], firsts_v)
    pltpu.sync_copy(lasts_hbm.at[:, pl.ds(b0, BPW)], lasts_v)

    def fire_gather(s, h, buf):
        idx = (firsts_v if h == 0 else lasts_v).at[lax.div(s, 2)]
        pltpu.async_copy(table_hbm.at[idx], rows_v.at[buf], gsem)

    def wait_gather(s, h, buf):
        idx = (firsts_v if h == 0 else lasts_v).at[lax.div(s, 2)]
        pltpu.make_async_copy(table_hbm.at[idx], rows_v.at[buf], gsem).wait()

    fire_gather(0, 0, 0)
    fire_gather(1, 1, 1)
    fire_gather(2, 0, 2)

    @pl.loop(0, NSLOT, step=NBUF)
    def _body(s0):
        for k in range(NBUF):
            s = s0 + k
            h = k % 2
            dst = out_hbm.at[lax.div(s, 2), pl.ds(b0, BPW),
                             pl.ds(h * DIM, DIM)]
            wait_gather(s, h, k)
            pltpu.async_copy(rows_v.at[k], dst, ssem)

            @pl.when(s >= 1)
            def _():
                pltpu.make_async_copy(rows_v.at[k], dst, ssem).wait()

            @pl.when(s + 3 < NSLOT)
            def _():
                fire_gather(s + 3, 1 - h, (k + 3) % NBUF)

    dst0 = out_hbm.at[0, pl.ds(b0, BPW), pl.ds(0, DIM)]
    pltpu.make_async_copy(rows_v.at[0], dst0, ssem).wait()


def kernel(first_last_ids, table):
    ids = first_last_ids.astype(jnp.int32)
    firsts_t = jnp.transpose(ids[..., 0])
    lasts_t = jnp.transpose(ids[..., 1])
    out = _gather_rows(firsts_t, lasts_t, table)
    return jnp.transpose(out, (1, 0, 2))


# speedup vs baseline: 1.0235x; 1.0235x over previous
import functools

import jax
import jax.numpy as jnp
from jax import lax
from jax.experimental import pallas as pl
from jax.experimental.pallas import tpu as pltpu
from jax.experimental.pallas import tpu_sc as plsc

VOCAB = 100000
DIM = 128
BATCH = 4096
SEQ = 50

_INFO = plsc.get_sparse_core_info()
NC = _INFO.num_cores
NS = _INFO.num_subcores
NW = NC * NS

BPW = BATCH // NW
NSLOT = 2 * SEQ
NBUF = 4


@functools.partial(
    pl.kernel,
    out_type=jax.ShapeDtypeStruct((SEQ, BATCH, 2 * DIM), jnp.float32),
    mesh=plsc.VectorSubcoreMesh(core_axis_name="c", subcore_axis_name="s"),
    scratch_types=[
        pltpu.VMEM((NSLOT, BPW), jnp.int32),
        pltpu.VMEM((NBUF, BPW, DIM), jnp.float32),
        pltpu.SemaphoreType.DMA,
        pltpu.SemaphoreType.DMA,
    ],
)
def _gather_rows(idx_hbm, table_hbm, out_hbm, idx_v, rows_v, gsem, ssem):
    wid = lax.axis_index("s") * NC + lax.axis_index("c")
    b0 = wid * BPW
    pltpu.sync_copy(idx_hbm.at[:, pl.ds(b0, BPW)], idx_v)

    def fire_gather(s, buf):
        pltpu.async_copy(table_hbm.at[idx_v.at[s]], rows_v.at[buf], gsem)

    def wait_gather(s, buf):
        pltpu.make_async_copy(table_hbm.at[idx_v.at[s]],
                              rows_v.at[buf], gsem).wait()

    fire_gather(0, 0)
    fire_gather(1, 1)
    fire_gather(2, 2)

    @pl.loop(0, NSLOT, step=NBUF)
    def _body(s0):
        for k in range(NBUF):
            s = s0 + k
            h = k % 2
            dst = out_hbm.at[lax.div(s, 2), pl.ds(b0, BPW),
                             pl.ds(h * DIM, DIM)]
            wait_gather(s, k)
            pltpu.async_copy(rows_v.at[k], dst, ssem)

            @pl.when(s >= 1)
            def _():
                pltpu.make_async_copy(rows_v.at[k], dst, ssem).wait()

            @pl.when(s + 3 < NSLOT)
            def _():
                fire_gather(s + 3, (k + 3) % NBUF)

    dst0 = out_hbm.at[0, pl.ds(b0, BPW), pl.ds(0, DIM)]
    pltpu.make_async_copy(rows_v.at[0], dst0, ssem).wait()


def kernel(first_last_ids, table):
    ids = first_last_ids.astype(jnp.int32)
    idx = jnp.transpose(ids, (1, 2, 0)).reshape(NSLOT, BATCH)
    out = _gather_rows(idx, table)
    return jnp.transpose(out, (1, 0, 2))
